# baseline (speedup 1.0000x reference)
import functools

import jax
import jax.numpy as jnp
from jax import lax
from jax.experimental import pallas as pl
from jax.experimental.pallas import tpu as pltpu
from jax.experimental.pallas import tpu_sc as plsc

_LOG2E = 1.4426950408889634
_LN2 = 0.6931471805599453

_NUM_WORKERS = 32


def _gather_target_sc(xflat, target, c):
    n = target.shape[0]
    per_w = n // _NUM_WORKERS
    mesh = plsc.VectorSubcoreMesh(core_axis_name="c", subcore_axis_name="s")

    @functools.partial(
        pl.kernel,
        mesh=mesh,
        out_type=jax.ShapeDtypeStruct((n,), jnp.float32),
        scratch_types=[
            pltpu.VMEM((per_w,), jnp.int32),
            pltpu.VMEM((per_w,), jnp.float32),
            pltpu.SemaphoreType.DMA,
        ],
    )
    def k(x_hbm, t_hbm, out_hbm, idx_v, val_v, sem):
        wid = lax.axis_index("s") * 2 + lax.axis_index("c")
        base = wid * per_w
        pltpu.sync_copy(t_hbm.at[pl.ds(base, per_w)], idx_v)
        lane = lax.broadcasted_iota(jnp.int32, (16,), 0)
        for j in range(per_w // 16):
            sl = pl.ds(j * 16, 16)
            idx_v[sl] = idx_v[sl] + (base + j * 16 + lane) * c
        pltpu.async_copy(x_hbm.at[idx_v], val_v, sem).wait()
        pltpu.sync_copy(val_v, out_hbm.at[pl.ds(base, per_w)])

    return k(xflat, target)


def _loss_body(x_ref, xt_ref, o_ref):
    x = x_ref[...]
    xt = xt_ref[...]
    ninf = jnp.float32(-jnp.inf)

    m = jnp.max(x, axis=1, keepdims=True)
    z = (x - m) * _LOG2E
    e = jnp.exp2(z)
    s = jnp.sum(e, axis=1, keepdims=True)
    zt = (xt - m) * _LOG2E

    zc = jnp.where(z < zt, z, ninf)
    t1 = jnp.max(zc, axis=1, keepdims=True)
    lt1 = zc < t1
    c1 = jnp.sum(jnp.where(lt1, 0.0, 1.0), axis=1, keepdims=True)
    t2 = jnp.max(jnp.where(lt1, zc, ninf), axis=1, keepdims=True)
    t2 = jnp.where(c1 >= 2.0, t1, t2)

    log2s = jnp.log2(s)
    logpk = (zt - log2s) * _LN2
    pk = jnp.exp2(zt - log2s)
    p1 = jnp.exp2(t1 - log2s)
    p2 = jnp.exp2(t2 - log2s)
    d = 1.0 - pk + p1 + p2
    blk = jnp.sum(-(d * d) * logpk)

    @pl.when(pl.program_id(0) == 0)
    def _init():
        o_ref[0, 0] = 0.0

    o_ref[0, 0] += blk


@functools.partial(jax.jit, static_argnames=("block_rows",))
def _dual_focal_loss(x, target, block_rows=1024):
    n, c = x.shape
    nb = n // block_rows
    xt = _gather_target_sc(x.reshape(n * c), target, c).reshape(n, 1)
    out = pl.pallas_call(
        _loss_body,
        grid=(nb,),
        in_specs=[
            pl.BlockSpec((block_rows, c), lambda i: (i, 0)),
            pl.BlockSpec((block_rows, 1), lambda i: (i, 0)),
        ],
        out_specs=pl.BlockSpec(memory_space=pltpu.SMEM),
        out_shape=jax.ShapeDtypeStruct((1, 1), jnp.float32),
    )(x, xt)
    return out[0, 0]


def kernel(input, target):
    return _dual_focal_loss(input, target)


# speedup vs baseline: 1.5213x; 1.5213x over previous
import functools

import jax
import jax.numpy as jnp
from jax import lax
from jax.experimental import pallas as pl
from jax.experimental.pallas import tpu as pltpu

_LOG2E = 1.4426950408889634
_LN2 = 0.6931471805599453


def _loss_body(x_ref, t_ref, o_ref):
    x = x_ref[...]
    t = t_ref[...]
    ninf = jnp.float32(-jnp.inf)

    cid = lax.broadcasted_iota(jnp.int32, x.shape, 1)
    xt = jnp.max(jnp.where(cid == t, x, ninf), axis=1, keepdims=True)

    m = jnp.max(x, axis=1, keepdims=True)
    z = (x - m) * _LOG2E
    e = jnp.exp2(z)
    s = jnp.sum(e, axis=1, keepdims=True)
    zt = (xt - m) * _LOG2E

    zc = jnp.where(z < zt, z, ninf)
    t1 = jnp.max(zc, axis=1, keepdims=True)
    lt1 = zc < t1
    c1 = jnp.sum(jnp.where(lt1, 0.0, 1.0), axis=1, keepdims=True)
    t2 = jnp.max(jnp.where(lt1, zc, ninf), axis=1, keepdims=True)
    t2 = jnp.where(c1 >= 2.0, t1, t2)

    log2s = jnp.log2(s)
    logpk = (zt - log2s) * _LN2
    pk = jnp.exp2(zt - log2s)
    p1 = jnp.exp2(t1 - log2s)
    p2 = jnp.exp2(t2 - log2s)
    d = 1.0 - pk + p1 + p2
    blk = jnp.sum(-(d * d) * logpk)

    @pl.when(pl.program_id(0) == 0)
    def _init():
        o_ref[0, 0] = 0.0

    o_ref[0, 0] += blk


@functools.partial(jax.jit, static_argnames=("block_rows",))
def _dual_focal_loss(x, target, block_rows=256):
    n, c = x.shape
    nb = n // block_rows
    out = pl.pallas_call(
        _loss_body,
        grid=(nb,),
        in_specs=[
            pl.BlockSpec((block_rows, c), lambda i: (i, 0)),
            pl.BlockSpec((block_rows, 1), lambda i: (i, 0)),
        ],
        out_specs=pl.BlockSpec(memory_space=pltpu.SMEM),
        out_shape=jax.ShapeDtypeStruct((1, 1), jnp.float32),
    )(x, target.reshape(n, 1))
    return out[0, 0]


def kernel(input, target):
    return _dual_focal_loss(input, target)


# speedup vs baseline: 1.7381x; 1.1426x over previous
import functools

import jax
import jax.numpy as jnp
from jax import lax
from jax.experimental import pallas as pl
from jax.experimental.pallas import tpu as pltpu

_LOG2E = 1.4426950408889634
_LN2 = 0.6931471805599453


def _loss_body(x_ref, t_ref, o_ref):
    x = x_ref[...]
    t = t_ref[...]
    ninf = jnp.float32(-jnp.inf)

    cid = lax.broadcasted_iota(jnp.int32, x.shape, 1)
    xt = jnp.max(jnp.where(cid == t, x, ninf), axis=1, keepdims=True)

    m = jnp.max(x, axis=1, keepdims=True)
    z = (x - m) * _LOG2E
    e = jnp.exp2(z)
    s = jnp.sum(e, axis=1, keepdims=True)
    zt = (xt - m) * _LOG2E

    zc = jnp.where(z < zt, z, ninf)
    t1 = jnp.max(zc, axis=1, keepdims=True)
    lt1 = zc < t1
    c1 = jnp.sum(jnp.where(lt1, 0.0, 1.0), axis=1, keepdims=True)
    t2 = jnp.max(jnp.where(lt1, zc, ninf), axis=1, keepdims=True)
    t2 = jnp.where(c1 >= 2.0, t1, t2)

    log2s = jnp.log2(s)
    logpk = (zt - log2s) * _LN2
    pk = jnp.exp2(zt - log2s)
    p1 = jnp.exp2(t1 - log2s)
    p2 = jnp.exp2(t2 - log2s)
    d = 1.0 - pk + p1 + p2
    blk = jnp.sum(-(d * d) * logpk)

    @pl.when(pl.program_id(0) == 0)
    def _init():
        o_ref[0, 0] = 0.0

    o_ref[0, 0] += blk


@functools.partial(jax.jit, static_argnames=("block_rows",))
def _dual_focal_loss(x, target, block_rows=512):
    n, c = x.shape
    nb = n // block_rows
    out = pl.pallas_call(
        _loss_body,
        grid=(nb,),
        in_specs=[
            pl.BlockSpec((block_rows, c), lambda i: (i, 0)),
            pl.BlockSpec((block_rows, 1), lambda i: (i, 0)),
        ],
        out_specs=pl.BlockSpec(memory_space=pltpu.SMEM),
        out_shape=jax.ShapeDtypeStruct((1, 1), jnp.float32),
    )(x, target.reshape(n, 1))
    return out[0, 0]


def kernel(input, target):
    return _dual_focal_loss(input, target)


# speedup vs baseline: 1.8211x; 1.0477x over previous
import functools

import jax
import jax.numpy as jnp
from jax import lax
from jax.experimental import pallas as pl
from jax.experimental.pallas import tpu as pltpu

_LOG2E = 1.4426950408889634
_LN2 = 0.6931471805599453


def _loss_body(x_ref, t_ref, o_ref):
    x = x_ref[...]
    t = t_ref[...]
    ninf = jnp.float32(-jnp.inf)

    cid = lax.broadcasted_iota(jnp.int32, x.shape, 1)
    xt = jnp.max(jnp.where(cid == t, x, ninf), axis=1, keepdims=True)

    m = jnp.max(x, axis=1, keepdims=True)
    z = (x - m) * _LOG2E
    e = jnp.exp2(z)
    s = jnp.sum(e, axis=1, keepdims=True)
    zt = (xt - m) * _LOG2E

    zc = jnp.where(z < zt, z, ninf)
    t1 = jnp.max(zc, axis=1, keepdims=True)
    lt1 = zc < t1
    c1 = jnp.sum(jnp.where(lt1, 0.0, 1.0), axis=1, keepdims=True)
    t2 = jnp.max(jnp.where(lt1, zc, ninf), axis=1, keepdims=True)
    t2 = jnp.where(c1 >= 2.0, t1, t2)

    log2s = jnp.log2(s)
    logpk = (zt - log2s) * _LN2
    pk = jnp.exp2(zt - log2s)
    p1 = jnp.exp2(t1 - log2s)
    p2 = jnp.exp2(t2 - log2s)
    d = 1.0 - pk + p1 + p2
    blk = jnp.sum(-(d * d) * logpk)

    @pl.when(pl.program_id(0) == 0)
    def _init():
        o_ref[0, 0] = 0.0

    o_ref[0, 0] += blk


@functools.partial(jax.jit, static_argnames=("block_rows",))
def _dual_focal_loss(x, target, block_rows=1024):
    n, c = x.shape
    nb = n // block_rows
    out = pl.pallas_call(
        _loss_body,
        grid=(nb,),
        in_specs=[
            pl.BlockSpec((block_rows, c), lambda i: (i, 0)),
            pl.BlockSpec((block_rows, 1), lambda i: (i, 0)),
        ],
        out_specs=pl.BlockSpec(memory_space=pltpu.SMEM),
        out_shape=jax.ShapeDtypeStruct((1, 1), jnp.float32),
    )(x, target.reshape(n, 1))
    return out[0, 0]


def kernel(input, target):
    return _dual_focal_loss(input, target)
